# speedup vs baseline: 4.0090x; 4.0090x over previous
import functools

import jax
import jax.numpy as jnp
from jax import lax
from jax.experimental import pallas as pl
from jax.experimental.pallas import tpu as pltpu
from jax.experimental.pallas import tpu_sc as plsc

_LOG_REG_C = 2.5e-05
_LEAF_COEF = 10.0

_NTRANS = 50000
_N_PARENTS = 800000
_N_STEMS = 400000
_NCOLS = 105

_NW = 32
_CHUNK_A = _N_PARENTS // _NW
_N_STEMS_PAD = 400384
_CHUNK_B = _N_STEMS_PAD // _NW

_R_BLK = 4000
_GRID1 = _N_STEMS // _R_BLK
_Q_BLK = _N_PARENTS // _GRID1 // 1000


def _tc1_body(stem_ref, qsa_ref, v_ref, expq_ref):
    v_ref[...] = jnp.sum(jnp.exp(stem_ref[...]), axis=1, keepdims=True)
    expq_ref[...] = jnp.exp(qsa_ref[...])


def _tc1(stem_out_s, qsa_p):
    qsa2 = qsa_p.reshape(_N_PARENTS // 1000, 1000)
    v, expq = pl.pallas_call(
        _tc1_body,
        grid=(_GRID1,),
        in_specs=[
            pl.BlockSpec((_R_BLK, _NCOLS), lambda i: (i, 0)),
            pl.BlockSpec((_Q_BLK, 1000), lambda i: (i, 0)),
        ],
        out_specs=[
            pl.BlockSpec((_R_BLK, 1), lambda i: (i, 0)),
            pl.BlockSpec((_Q_BLK, 1000), lambda i: (i, 0)),
        ],
        out_shape=[
            jax.ShapeDtypeStruct((_N_STEMS, 1), jnp.float32),
            jax.ShapeDtypeStruct((_N_PARENTS // 1000, 1000), jnp.float32),
        ],
    )(stem_out_s, qsa2)
    return v.reshape(_N_STEMS), expq.reshape(_N_PARENTS)


def _sc_body(expq_hbm, pb_hbm, v_hbm, sb_hbm, zeros_hbm, out_a_hbm, out_b_hbm,
             vals_a, idx_a, vals_b, idx_b, acc_a, acc_b):
    c = lax.axis_index("c")
    s = lax.axis_index("s")
    wid = c * 16 + s

    @pl.when(s == 0)
    def _():
        pltpu.sync_copy(zeros_hbm, acc_a)
        pltpu.sync_copy(zeros_hbm, acc_b)

    plsc.subcore_barrier()

    pltpu.sync_copy(expq_hbm.at[wid], vals_a)
    pltpu.sync_copy(pb_hbm.at[wid], idx_a)
    pltpu.sync_copy(v_hbm.at[wid], vals_b)
    pltpu.sync_copy(sb_hbm.at[wid], idx_b)

    pltpu.sync_copy(vals_a, acc_a.at[idx_a], add=True)
    pltpu.sync_copy(vals_b, acc_b.at[idx_b], add=True)

    plsc.subcore_barrier()

    @pl.when(s == 0)
    def _():
        pltpu.sync_copy(acc_a, out_a_hbm.at[c])
        pltpu.sync_copy(acc_b, out_b_hbm.at[c])


_sc_seg = pl.kernel(
    _sc_body,
    mesh=plsc.VectorSubcoreMesh(core_axis_name="c", subcore_axis_name="s"),
    out_type=[
        jax.ShapeDtypeStruct((2, _NTRANS), jnp.float32),
        jax.ShapeDtypeStruct((2, _NTRANS), jnp.float32),
    ],
    scratch_types=[
        pltpu.VMEM((_CHUNK_A,), jnp.float32),
        pltpu.VMEM((_CHUNK_A,), jnp.int32),
        pltpu.VMEM((_CHUNK_B,), jnp.float32),
        pltpu.VMEM((_CHUNK_B,), jnp.int32),
        pltpu.VMEM_SHARED((_NTRANS,), jnp.float32),
        pltpu.VMEM_SHARED((_NTRANS,), jnp.float32),
    ],
)


def _tc2_body(pa_ref, pb_ref, mol_ref, r_ref, d_ref, loss_ref, term_ref, flow_ref):
    exp_inflow = jnp.sum(pa_ref[...], axis=0, keepdims=True)
    inflow = jnp.log(exp_inflow + _LOG_REG_C)
    exp_outflow = jnp.sum(pb_ref[...], axis=0, keepdims=True) + jnp.exp(mol_ref[...])
    dd = d_ref[...]
    opr = jnp.log(_LOG_REG_C + r_ref[...] + exp_outflow * (1.0 - dd))
    losses = (inflow - opr) ** 2
    term = jnp.sum(losses * dd) / (jnp.sum(dd) + 1e-20)
    flow = jnp.sum(losses * (1.0 - dd)) / (jnp.sum(1.0 - dd) + 1e-20)
    loss_ref[0, 0] = term * _LEAF_COEF + flow
    term_ref[0, 0] = term
    flow_ref[0, 0] = flow


def _tc2(part_a, part_b, mol2, r2, d2):
    return pl.pallas_call(
        _tc2_body,
        out_specs=[
            pl.BlockSpec(memory_space=pltpu.SMEM),
            pl.BlockSpec(memory_space=pltpu.SMEM),
            pl.BlockSpec(memory_space=pltpu.SMEM),
        ],
        out_shape=[
            jax.ShapeDtypeStruct((1, 1), jnp.float32),
            jax.ShapeDtypeStruct((1, 1), jnp.float32),
            jax.ShapeDtypeStruct((1, 1), jnp.float32),
        ],
    )(part_a, part_b, mol2, r2, d2)


def kernel(stem_out_s, mol_out_s, qsa_p, r, d, pb, stem_batch):
    v, expq = _tc1(stem_out_s, qsa_p)

    pb_i = pb.astype(jnp.int32).reshape(_NW, _CHUNK_A)
    expq2 = expq.reshape(_NW, _CHUNK_A)
    pad_n = _N_STEMS_PAD - _N_STEMS
    v2 = jnp.concatenate([v, jnp.zeros((pad_n,), jnp.float32)]).reshape(_NW, _CHUNK_B)
    sb2 = jnp.concatenate(
        [stem_batch.astype(jnp.int32), jnp.zeros((pad_n,), jnp.int32)]
    ).reshape(_NW, _CHUNK_B)
    zeros = jnp.zeros((_NTRANS,), jnp.float32)

    part_a, part_b = _sc_seg(expq2, pb_i, v2, sb2, zeros)

    mol2 = mol_out_s.reshape(1, _NTRANS)
    r2 = r.reshape(1, _NTRANS)
    d2 = d.reshape(1, _NTRANS)
    loss, term, flow = _tc2(part_a, part_b, mol2, r2, d2)
    return (loss[0, 0], term[0, 0], flow[0, 0])


# speedup vs baseline: 13.2738x; 3.3110x over previous
import functools

import jax
import jax.numpy as jnp
from jax import lax
from jax.experimental import pallas as pl
from jax.experimental.pallas import tpu as pltpu
from jax.experimental.pallas import tpu_sc as plsc

_LOG_REG_C = 2.5e-05
_LEAF_COEF = 10.0

_NTRANS = 50000
_N_PARENTS = 800000
_N_STEMS = 400000
_NCOLS = 105

_NW = 32
_CHUNK_A = _N_PARENTS // _NW
_N_STEMS_PAD = 400384
_CHUNK_B = _N_STEMS_PAD // _NW

_C_BLK = 16000
_GRID1 = _N_STEMS // _C_BLK
_Q_BLK = _N_PARENTS // 1000 // _GRID1


def _tc1_body(stem_ref, qsa_ref, v_ref, expq_ref):
    v_ref[...] = jnp.sum(jnp.exp(stem_ref[...]), axis=0, keepdims=True)
    expq_ref[...] = jnp.exp(qsa_ref[...])


def _tc1(stem_t, qsa2):
    v, expq = pl.pallas_call(
        _tc1_body,
        grid=(_GRID1,),
        in_specs=[
            pl.BlockSpec((_NCOLS, _C_BLK), lambda i: (0, i)),
            pl.BlockSpec((_Q_BLK, 1000), lambda i: (i, 0)),
        ],
        out_specs=[
            pl.BlockSpec((1, _C_BLK), lambda i: (0, i)),
            pl.BlockSpec((_Q_BLK, 1000), lambda i: (i, 0)),
        ],
        out_shape=[
            jax.ShapeDtypeStruct((1, _N_STEMS), jnp.float32),
            jax.ShapeDtypeStruct((_N_PARENTS // 1000, 1000), jnp.float32),
        ],
    )(stem_t, qsa2)
    return v.reshape(_N_STEMS), expq.reshape(_N_PARENTS)


def _sc_body(expq_hbm, pb_hbm, v_hbm, sb_hbm, zeros_hbm, out_a_hbm, out_b_hbm,
             vals_a, idx_a, vals_b, idx_b, acc_a, acc_b):
    c = lax.axis_index("c")
    s = lax.axis_index("s")
    wid = c * 16 + s

    @pl.when(s == 0)
    def _():
        pltpu.sync_copy(zeros_hbm, acc_a)
        pltpu.sync_copy(zeros_hbm, acc_b)

    plsc.subcore_barrier()

    pltpu.sync_copy(expq_hbm.at[wid], vals_a)
    pltpu.sync_copy(pb_hbm.at[wid], idx_a)
    pltpu.sync_copy(v_hbm.at[wid], vals_b)
    pltpu.sync_copy(sb_hbm.at[wid], idx_b)

    pltpu.sync_copy(vals_a, acc_a.at[idx_a], add=True)
    pltpu.sync_copy(vals_b, acc_b.at[idx_b], add=True)

    plsc.subcore_barrier()

    @pl.when(s == 0)
    def _():
        pltpu.sync_copy(acc_a, out_a_hbm.at[c])
        pltpu.sync_copy(acc_b, out_b_hbm.at[c])


_sc_seg = pl.kernel(
    _sc_body,
    mesh=plsc.VectorSubcoreMesh(core_axis_name="c", subcore_axis_name="s"),
    out_type=[
        jax.ShapeDtypeStruct((2, _NTRANS), jnp.float32),
        jax.ShapeDtypeStruct((2, _NTRANS), jnp.float32),
    ],
    scratch_types=[
        pltpu.VMEM((_CHUNK_A,), jnp.float32),
        pltpu.VMEM((_CHUNK_A,), jnp.int32),
        pltpu.VMEM((_CHUNK_B,), jnp.float32),
        pltpu.VMEM((_CHUNK_B,), jnp.int32),
        pltpu.VMEM_SHARED((_NTRANS,), jnp.float32),
        pltpu.VMEM_SHARED((_NTRANS,), jnp.float32),
    ],
)


def _tc2_body(pa_ref, pb_ref, mol_ref, r_ref, d_ref, loss_ref, term_ref, flow_ref):
    exp_inflow = jnp.sum(pa_ref[...], axis=0, keepdims=True)
    inflow = jnp.log(exp_inflow + _LOG_REG_C)
    exp_outflow = jnp.sum(pb_ref[...], axis=0, keepdims=True) + jnp.exp(mol_ref[...])
    dd = d_ref[...]
    opr = jnp.log(_LOG_REG_C + r_ref[...] + exp_outflow * (1.0 - dd))
    losses = (inflow - opr) ** 2
    term = jnp.sum(losses * dd) / (jnp.sum(dd) + 1e-20)
    flow = jnp.sum(losses * (1.0 - dd)) / (jnp.sum(1.0 - dd) + 1e-20)
    loss_ref[0, 0] = term * _LEAF_COEF + flow
    term_ref[0, 0] = term
    flow_ref[0, 0] = flow


def _tc2(part_a, part_b, mol2, r2, d2):
    return pl.pallas_call(
        _tc2_body,
        out_specs=[
            pl.BlockSpec(memory_space=pltpu.SMEM),
            pl.BlockSpec(memory_space=pltpu.SMEM),
            pl.BlockSpec(memory_space=pltpu.SMEM),
        ],
        out_shape=[
            jax.ShapeDtypeStruct((1, 1), jnp.float32),
            jax.ShapeDtypeStruct((1, 1), jnp.float32),
            jax.ShapeDtypeStruct((1, 1), jnp.float32),
        ],
    )(part_a, part_b, mol2, r2, d2)


def kernel(stem_out_s, mol_out_s, qsa_p, r, d, pb, stem_batch):
    v, expq = _tc1(stem_out_s.T, qsa_p.reshape(_N_PARENTS // 1000, 1000))

    pb_i = pb.astype(jnp.int32).reshape(_NW, _CHUNK_A)
    expq2 = expq.reshape(_NW, _CHUNK_A)
    pad_n = _N_STEMS_PAD - _N_STEMS
    v2 = jnp.concatenate([v, jnp.zeros((pad_n,), jnp.float32)]).reshape(_NW, _CHUNK_B)
    sb2 = jnp.concatenate(
        [stem_batch.astype(jnp.int32), jnp.zeros((pad_n,), jnp.int32)]
    ).reshape(_NW, _CHUNK_B)
    zeros = jnp.zeros((_NTRANS,), jnp.float32)

    part_a, part_b = _sc_seg(expq2, pb_i, v2, sb2, zeros)

    mol2 = mol_out_s.reshape(1, _NTRANS)
    r2 = r.reshape(1, _NTRANS)
    d2 = d.reshape(1, _NTRANS)
    loss, term, flow = _tc2(part_a, part_b, mol2, r2, d2)
    return (loss[0, 0], term[0, 0], flow[0, 0])


# speedup vs baseline: 15.8660x; 1.1953x over previous
import jax
import jax.numpy as jnp
from jax import lax
from jax.experimental import pallas as pl
from jax.experimental.pallas import tpu as pltpu
from jax.experimental.pallas import tpu_sc as plsc

_LOG_REG_C = 2.5e-05
_LEAF_COEF = 10.0

_NTRANS = 50000
_N_PARENTS = 800000
_N_STEMS = 400000
_NCOLS = 105

_NW = 32
_NP_PAD = 800256
_CHUNK_A = _NP_PAD // _NW
_NS_PAD = 400384
_CHUNK_B = _NS_PAD // _NW

_C_BLK = 16000
_GRID1 = _N_STEMS // _C_BLK


def _tc1_body(stem_ref, v_ref):
    v_ref[...] = jnp.sum(jnp.exp(stem_ref[...]), axis=0, keepdims=True)


def _tc1(stem_t):
    return pl.pallas_call(
        _tc1_body,
        grid=(_GRID1,),
        in_specs=[pl.BlockSpec((_NCOLS, _C_BLK), lambda i: (0, i))],
        out_specs=pl.BlockSpec((1, _C_BLK), lambda i: (0, i)),
        out_shape=jax.ShapeDtypeStruct((1, _N_STEMS), jnp.float32),
    )(stem_t)


def _sca_body(qsa_hbm, pb_hbm, zeros_hbm, out_hbm, vals, idx, acc):
    c = lax.axis_index("c")
    s = lax.axis_index("s")
    wid = c * 16 + s

    @pl.when(s == 0)
    def _():
        pltpu.sync_copy(zeros_hbm, acc)

    pltpu.sync_copy(qsa_hbm.at[pl.ds(wid * _CHUNK_A, _CHUNK_A)], vals)
    pltpu.sync_copy(pb_hbm.at[pl.ds(wid * _CHUNK_A, _CHUNK_A)], idx)

    def _exp_step(i, carry):
        sl = pl.ds(i * 16, 16)
        vals[sl] = jnp.exp(vals[sl])
        return carry

    lax.fori_loop(0, _CHUNK_A // 16, _exp_step, 0)

    plsc.subcore_barrier()
    pltpu.sync_copy(vals, acc.at[idx], add=True)
    plsc.subcore_barrier()

    @pl.when(s == 0)
    def _():
        pltpu.sync_copy(acc, out_hbm.at[c])


_sc_a = pl.kernel(
    _sca_body,
    mesh=plsc.VectorSubcoreMesh(core_axis_name="c", subcore_axis_name="s"),
    out_type=jax.ShapeDtypeStruct((2, _NTRANS), jnp.float32),
    scratch_types=[
        pltpu.VMEM((_CHUNK_A,), jnp.float32),
        pltpu.VMEM((_CHUNK_A,), jnp.int32),
        pltpu.VMEM_SHARED((_NTRANS,), jnp.float32),
    ],
)


def _scb_body(v_hbm, sb_hbm, zeros_hbm, out_hbm, vals, idx, acc):
    c = lax.axis_index("c")
    s = lax.axis_index("s")
    wid = c * 16 + s

    @pl.when(s == 0)
    def _():
        pltpu.sync_copy(zeros_hbm, acc)

    pltpu.sync_copy(v_hbm.at[pl.ds(wid * _CHUNK_B, _CHUNK_B)], vals)
    pltpu.sync_copy(sb_hbm.at[pl.ds(wid * _CHUNK_B, _CHUNK_B)], idx)

    plsc.subcore_barrier()
    pltpu.sync_copy(vals, acc.at[idx], add=True)
    plsc.subcore_barrier()

    @pl.when(s == 0)
    def _():
        pltpu.sync_copy(acc, out_hbm.at[c])


_sc_b = pl.kernel(
    _scb_body,
    mesh=plsc.VectorSubcoreMesh(core_axis_name="c", subcore_axis_name="s"),
    out_type=jax.ShapeDtypeStruct((2, _NTRANS), jnp.float32),
    scratch_types=[
        pltpu.VMEM((_CHUNK_B,), jnp.float32),
        pltpu.VMEM((_CHUNK_B,), jnp.int32),
        pltpu.VMEM_SHARED((_NTRANS,), jnp.float32),
    ],
)


def _tc2_body(pa_ref, pb_ref, mol_ref, r_ref, d_ref, loss_ref, term_ref, flow_ref):
    exp_inflow = jnp.sum(pa_ref[...], axis=0, keepdims=True)
    inflow = jnp.log(exp_inflow + _LOG_REG_C)
    exp_outflow = jnp.sum(pb_ref[...], axis=0, keepdims=True) + jnp.exp(mol_ref[...])
    dd = d_ref[...]
    opr = jnp.log(_LOG_REG_C + r_ref[...] + exp_outflow * (1.0 - dd))
    losses = (inflow - opr) ** 2
    term = jnp.sum(losses * dd) / (jnp.sum(dd) + 1e-20)
    flow = jnp.sum(losses * (1.0 - dd)) / (jnp.sum(1.0 - dd) + 1e-20)
    loss_ref[0, 0] = term * _LEAF_COEF + flow
    term_ref[0, 0] = term
    flow_ref[0, 0] = flow


def _tc2(part_a, part_b, mol2, r2, d2):
    return pl.pallas_call(
        _tc2_body,
        out_specs=[
            pl.BlockSpec(memory_space=pltpu.SMEM),
            pl.BlockSpec(memory_space=pltpu.SMEM),
            pl.BlockSpec(memory_space=pltpu.SMEM),
        ],
        out_shape=[
            jax.ShapeDtypeStruct((1, 1), jnp.float32),
            jax.ShapeDtypeStruct((1, 1), jnp.float32),
            jax.ShapeDtypeStruct((1, 1), jnp.float32),
        ],
    )(part_a, part_b, mol2, r2, d2)


def kernel(stem_out_s, mol_out_s, qsa_p, r, d, pb, stem_batch):
    qsa_pad = jnp.concatenate(
        [qsa_p, jnp.full((_NP_PAD - _N_PARENTS,), -1e30, jnp.float32)]
    )
    pb_pad = jnp.concatenate(
        [pb.astype(jnp.int32), jnp.zeros((_NP_PAD - _N_PARENTS,), jnp.int32)]
    )
    zeros = jnp.zeros((_NTRANS,), jnp.float32)
    part_a = _sc_a(qsa_pad, pb_pad, zeros)

    v = _tc1(stem_out_s.T).reshape(_N_STEMS)

    v_pad = jnp.concatenate([v, jnp.zeros((_NS_PAD - _N_STEMS,), jnp.float32)])
    sb_pad = jnp.concatenate(
        [stem_batch.astype(jnp.int32), jnp.zeros((_NS_PAD - _N_STEMS,), jnp.int32)]
    )
    part_b = _sc_b(v_pad, sb_pad, zeros)

    mol2 = mol_out_s.reshape(1, _NTRANS)
    r2 = r.reshape(1, _NTRANS)
    d2 = d.reshape(1, _NTRANS)
    loss, term, flow = _tc2(part_a, part_b, mol2, r2, d2)
    return (loss[0, 0], term[0, 0], flow[0, 0])
